# baseline (speedup 1.0000x reference)
import jax
import jax.numpy as jnp
from jax import lax
from jax.experimental import pallas as pl
from jax.experimental.pallas import tpu as pltpu
from jax.experimental.pallas import tpu_sc as plsc

VOC_SIZE = 1000000
EMBED_DIM = 64
BATCH = 16384
HIST = 20

NC = 2
NS = 16
NW = NC * NS

CHUNK = 128
NCHUNKS = BATCH * HIST // CHUNK
CPW = NCHUNKS // NW
CBLK = BATCH // CHUNK


def _extract(G, par_ref, j, OUT):
    for lg in range(8):
        lvec = jnp.arange(16, dtype=jnp.int32) + 16 * lg
        pb = par_ref[j, pl.ds(16 * lg, 16)] * 64
        for r in range(8):
            for s in range(8):
                cvec = pb + (8 * r + s)
                val = plsc.load_gather(G, [lvec, cvec])
                OUT[r, s, pl.ds(16 * lg, 16)] = val


def _body(pidx_hbm, par_hbm, table_hbm, out_hbm, pv, qv, Ga, Gb, Oa, Ob,
          gsa, gsb, osa, osb):
    c = lax.axis_index("c")
    s = lax.axis_index("s")
    wid = s * NC + c
    k0 = wid * CPW
    pltpu.sync_copy(pidx_hbm.at[wid], pv)
    pltpu.sync_copy(par_hbm.at[wid], qv)

    def start_gather(j, G, sem):
        pltpu.async_copy(table_hbm.at[pv.at[j]], G, sem)

    def wait_gather(j, G, sem):
        pltpu.make_async_copy(table_hbm.at[pv.at[j]], G, sem).wait()

    def out_dst(j, r):
        k = k0 + j
        return out_hbm.at[k // CBLK, r, k % CBLK]

    def start_out(j, OUT, sem):
        for r in range(8):
            pltpu.async_copy(OUT.at[r], out_dst(j, r), sem)

    def wait_out(j, OUT, sem):
        for r in range(8):
            pltpu.make_async_copy(OUT.at[r], out_dst(j, r), sem).wait()

    start_gather(0, Ga, gsa)
    start_gather(1, Gb, gsb)

    def block(t, carry):
        for G, OUT, gs, os, off in ((Ga, Oa, gsa, osa, 0),
                                    (Gb, Ob, gsb, osb, 1)):
            j = 2 * t + off
            wait_gather(j, G, gs)

            @pl.when(t >= 1)
            def _():
                wait_out(j - 2, OUT, os)

            _extract(G, qv, j, OUT)

            @pl.when(j + 2 < CPW)
            def _():
                start_gather(j + 2, G, gs)

            start_out(j, OUT, os)
        return carry

    lax.fori_loop(0, CPW // 2, block, 0)
    wait_out(CPW - 2, Oa, osa)
    wait_out(CPW - 1, Ob, osb)


@jax.jit
def _lookup(pidx, par, table2):
    mesh = plsc.VectorSubcoreMesh(core_axis_name="c", subcore_axis_name="s")
    kfn = pl.kernel(
        _body,
        out_type=jax.ShapeDtypeStruct((HIST, 8, CBLK, 8, CHUNK), jnp.float32),
        mesh=mesh,
        scratch_types=[
            pltpu.VMEM((CPW, CHUNK), jnp.int32),
            pltpu.VMEM((CPW, CHUNK), jnp.int32),
            pltpu.VMEM((CHUNK, 128), jnp.float32),
            pltpu.VMEM((CHUNK, 128), jnp.float32),
            pltpu.VMEM((8, 8, CHUNK), jnp.float32),
            pltpu.VMEM((8, 8, CHUNK), jnp.float32),
            pltpu.SemaphoreType.DMA,
            pltpu.SemaphoreType.DMA,
            pltpu.SemaphoreType.DMA,
            pltpu.SemaphoreType.DMA,
        ],
        compiler_params=pltpu.CompilerParams(
            use_tc_tiling_on_sc=False, needs_layout_passes=False
        ),
    )
    return kfn(pidx, par, table2)


def kernel(inputs, embeddings):
    idx_t = inputs.astype(jnp.int32).T.reshape(NW, CPW, CHUNK)
    pidx = idx_t >> 1
    par = idx_t & 1
    table2 = embeddings.reshape(VOC_SIZE // 2, 128)
    out5 = _lookup(pidx, par, table2)
    return out5.transpose(2, 4, 0, 1, 3).reshape(BATCH, HIST, EMBED_DIM)


# speedup vs baseline: 1.2886x; 1.2886x over previous
import jax
import jax.numpy as jnp
from jax import lax
from jax.experimental import pallas as pl
from jax.experimental.pallas import tpu as pltpu
from jax.experimental.pallas import tpu_sc as plsc

VOC_SIZE = 1000000
EMBED_DIM = 64
BATCH = 16384
HIST = 20

NC = 2
NS = 16
NW = NC * NS

CHUNK = 128
NCHUNKS = BATCH * HIST // CHUNK
CPW = NCHUNKS // NW
CBLK = BATCH // CHUNK


def _transpose(G, OUT):
    for lg in range(8):
        lvec = jnp.arange(16, dtype=jnp.int32) + 16 * lg

        @plsc.parallel_loop(0, EMBED_DIM, step=1, unroll=8)
        def _(d):
            cvec = jnp.full((16,), 0, jnp.int32) + d
            OUT[d, pl.ds(16 * lg, 16)] = plsc.load_gather(G, [lvec, cvec])


def _body(idx_hbm, table_hbm, out_hbm, pv, Ga, Gb, Oa, Ob,
          gsa, gsb, osa, osb):
    c = lax.axis_index("c")
    s = lax.axis_index("s")
    wid = s * NC + c
    k0 = wid * CPW
    pltpu.sync_copy(idx_hbm.at[wid], pv)

    def start_gather(j, G, sem):
        pltpu.async_copy(table_hbm.at[pv.at[j]], G, sem)

    def wait_gather(j, G, sem):
        pltpu.make_async_copy(table_hbm.at[pv.at[j]], G, sem).wait()

    def out_dst(j, r):
        k = k0 + j
        return out_hbm.at[k // CBLK, r, k % CBLK]

    def start_out(j, OUT, sem):
        for r in range(8):
            pltpu.async_copy(OUT.at[pl.ds(8 * r, 8)], out_dst(j, r), sem)

    def wait_out(j, OUT, sem):
        for r in range(8):
            pltpu.make_async_copy(OUT.at[pl.ds(8 * r, 8)], out_dst(j, r),
                                  sem).wait()

    start_gather(0, Ga, gsa)
    start_gather(1, Gb, gsb)

    def block(t, carry):
        for G, OUT, gs, os, off in ((Ga, Oa, gsa, osa, 0),
                                    (Gb, Ob, gsb, osb, 1)):
            j = 2 * t + off
            wait_gather(j, G, gs)

            @pl.when(t >= 1)
            def _():
                wait_out(j - 2, OUT, os)

            _transpose(G, OUT)

            @pl.when(j + 2 < CPW)
            def _():
                start_gather(j + 2, G, gs)

            start_out(j, OUT, os)
        return carry

    lax.fori_loop(0, CPW // 2, block, 0)
    wait_out(CPW - 2, Oa, osa)
    wait_out(CPW - 1, Ob, osb)


@jax.jit
def _lookup(idx3, table2):
    table = table2.reshape(VOC_SIZE, EMBED_DIM)
    mesh = plsc.VectorSubcoreMesh(core_axis_name="c", subcore_axis_name="s")
    kfn = pl.kernel(
        _body,
        out_type=jax.ShapeDtypeStruct((HIST, 8, CBLK, 8, CHUNK), jnp.float32),
        mesh=mesh,
        scratch_types=[
            pltpu.VMEM((CPW, CHUNK), jnp.int32),
            pltpu.VMEM((CHUNK, EMBED_DIM), jnp.float32),
            pltpu.VMEM((CHUNK, EMBED_DIM), jnp.float32),
            pltpu.VMEM((EMBED_DIM, CHUNK), jnp.float32),
            pltpu.VMEM((EMBED_DIM, CHUNK), jnp.float32),
            pltpu.SemaphoreType.DMA,
            pltpu.SemaphoreType.DMA,
            pltpu.SemaphoreType.DMA,
            pltpu.SemaphoreType.DMA,
        ],
        compiler_params=pltpu.CompilerParams(
            use_tc_tiling_on_sc=False, needs_layout_passes=False
        ),
    )
    return kfn(idx3, table)


def kernel(inputs, embeddings):
    idx3 = inputs.astype(jnp.int32).T.reshape(NW, CPW, CHUNK)
    table2 = embeddings.reshape(VOC_SIZE // 2, 128)
    out5 = _lookup(idx3, table2)
    return out5.transpose(2, 4, 0, 1, 3).reshape(BATCH, HIST, EMBED_DIM)
